# baseline (speedup 1.0000x reference)
import functools

import jax
import jax.numpy as jnp
from jax import lax
from jax.experimental import pallas as pl
from jax.experimental.pallas import tpu as pltpu
from jax.experimental.pallas import tpu_sc as plsc

BATCH = 4096
SEQ_LEN = 200
VOCAB = 6
DIM = 64
TOTAL = BATCH * SEQ_LEN

_info = plsc.get_sparse_core_info()
_NC, _NS = _info.num_cores, _info.num_subcores
_NW = _NC * _NS
_PER_W = TOTAL // _NW
_CH = 640
_NCH = _PER_W // _CH
_NCH2 = _NCH // 2

_DNUMS = lax.GatherDimensionNumbers(
    offset_dims=(), collapsed_slice_dims=(0,), start_index_map=(0,))


def _dgather(src, idx):
    return lax.gather(src, idx[:, None], _DNUMS, slice_sizes=(1,),
                      mode=lax.GatherScatterMode.PROMISE_IN_BOUNDS)


def _make_sc_gather():
    mesh = plsc.VectorSubcoreMesh(core_axis_name="c", subcore_axis_name="s")

    @functools.partial(
        pl.kernel,
        mesh=mesh,
        compiler_params=pltpu.CompilerParams(needs_layout_passes=False),
        out_type=jax.ShapeDtypeStruct((DIM, TOTAL), jnp.float32),
        scratch_types=[
            pltpu.VMEM((VOCAB, DIM), jnp.float32),
            pltpu.VMEM((DIM, 16), jnp.float32),
            pltpu.VMEM((_CH,), jnp.int32),
            pltpu.VMEM((_CH,), jnp.int32),
            pltpu.VMEM((DIM, _CH), jnp.float32),
            pltpu.VMEM((DIM, _CH), jnp.float32),
        ] + [pltpu.SemaphoreType.DMA] * 4,
    )
    def gather_kernel(table_hbm, idx_hbm, out_hbm,
                      tab_v, ttab_v, idx0_v, idx1_v, b0_v, b1_v, *sems):
        isems = sems[0:2]
        ssems = sems[2:4]
        wid = lax.axis_index("s") * _NC + lax.axis_index("c")
        wbase = wid * _PER_W

        pltpu.sync_copy(table_hbm, tab_v)

        rows = jnp.minimum(lax.iota(jnp.int32, 16), VOCAB - 1)
        for d in range(DIM):
            cols = jnp.full((16,), d, jnp.int32)
            ttab_v[d, :] = plsc.load_gather(tab_v, [rows, cols])

        idxbufs = [idx0_v, idx1_v]
        blocks = [b0_v, b1_v]

        def idx_src(c):
            return idx_hbm.at[pl.ds(wbase + c * _CH, _CH)]

        def out_dst(c):
            return out_hbm.at[:, pl.ds(wbase + c * _CH, _CH)]

        def compute(b):
            ib = idxbufs[b]
            ob = blocks[b]

            @plsc.parallel_loop(0, _CH // 16, unroll=5)
            def _(j):
                idxv = ib[pl.ds(j * 16, 16)]
                vals = [_dgather(ttab_v[d, :], idxv) for d in range(DIM)]
                for d in range(DIM):
                    ob[d, pl.ds(j * 16, 16)] = vals[d]

        pltpu.async_copy(idx_src(0), idx0_v, isems[0])
        pltpu.async_copy(idx_src(1), idx1_v, isems[1])

        def step(c2, _):
            for b in range(2):
                c = 2 * c2 + b
                pltpu.make_async_copy(idx_src(c), idxbufs[b], isems[b]).wait()

                @pl.when(c2 >= 1)
                def _():
                    pltpu.make_async_copy(blocks[b], out_dst(c),
                                          ssems[b]).wait()

                compute(b)
                pltpu.async_copy(blocks[b], out_dst(c), ssems[b])

                @pl.when(c2 < _NCH2 - 1)
                def _():
                    pltpu.async_copy(idx_src(c + 2), idxbufs[b], isems[b])
            return 0

        lax.fori_loop(0, _NCH2, step, 0)
        pltpu.make_async_copy(blocks[0], out_dst(0), ssems[0]).wait()
        pltpu.make_async_copy(blocks[1], out_dst(1), ssems[1]).wait()

    return gather_kernel


_sc_gather = _make_sc_gather()


def kernel(indices, table):
    flat_idx = indices.reshape(-1).astype(jnp.int32)
    out_t = _sc_gather(table, flat_idx)
    return out_t.T


# speedup vs baseline: 2.0048x; 2.0048x over previous
import functools

import jax
import jax.numpy as jnp
from jax import lax
from jax.experimental import pallas as pl
from jax.experimental.pallas import tpu as pltpu
from jax.experimental.pallas import tpu_sc as plsc

BATCH = 4096
SEQ_LEN = 200
VOCAB = 6
DIM = 64
TOTAL = BATCH * SEQ_LEN

_info = plsc.get_sparse_core_info()
_NC, _NS = _info.num_cores, _info.num_subcores
_NW = _NC * _NS
_PER_W = TOTAL // _NW
_CH = 640
_NCH = _PER_W // _CH
_NCH2 = _NCH // 2

_DNUMS = lax.GatherDimensionNumbers(
    offset_dims=(), collapsed_slice_dims=(0,), start_index_map=(0,))


def _dgather(src, idx):
    return lax.gather(src, idx[:, None], _DNUMS, slice_sizes=(1,),
                      mode=lax.GatherScatterMode.PROMISE_IN_BOUNDS)


def _make_sc_gather():
    mesh = plsc.VectorSubcoreMesh(core_axis_name="c", subcore_axis_name="s")

    @functools.partial(
        pl.kernel,
        mesh=mesh,
        compiler_params=pltpu.CompilerParams(needs_layout_passes=False),
        out_type=jax.ShapeDtypeStruct((DIM, TOTAL), jnp.float32),
        scratch_types=[
            pltpu.VMEM((VOCAB, DIM), jnp.float32),
            pltpu.VMEM((DIM, 16), jnp.float32),
            pltpu.VMEM((_CH,), jnp.int32),
            pltpu.VMEM((_CH,), jnp.int32),
            pltpu.VMEM((DIM, _CH), jnp.float32),
            pltpu.VMEM((DIM, _CH), jnp.float32),
        ] + [pltpu.SemaphoreType.DMA] * 4,
    )
    def gather_kernel(table_hbm, idx_hbm, out_hbm,
                      tab_v, ttab_v, idx0_v, idx1_v, b0_v, b1_v, *sems):
        isems = sems[0:2]
        ssems = sems[2:4]
        wid = lax.axis_index("s") * _NC + lax.axis_index("c")
        wbase = wid * _PER_W

        pltpu.sync_copy(table_hbm, tab_v)

        rows = jnp.minimum(lax.iota(jnp.int32, 16), VOCAB - 1)
        for d in range(DIM):
            cols = jnp.full((16,), d, jnp.int32)
            ttab_v[d, :] = plsc.load_gather(tab_v, [rows, cols])

        idxbufs = [idx0_v, idx1_v]
        blocks = [b0_v, b1_v]

        def idx_src(c):
            return idx_hbm.at[pl.ds(wbase + c * _CH, _CH)]

        def out_dst(c):
            return out_hbm.at[:, pl.ds(wbase + c * _CH, _CH)]

        def compute(b):
            ib = idxbufs[b]
            ob = blocks[b]

            @plsc.parallel_loop(0, _CH // 16, unroll=2)
            def _(j):
                idxv = ib[pl.ds(j * 16, 16)]
                vals = [_dgather(ttab_v[d, :], idxv) for d in range(DIM)]
                for d in range(DIM):
                    ob[d, pl.ds(j * 16, 16)] = vals[d]

        pltpu.async_copy(idx_src(0), idx0_v, isems[0])
        pltpu.async_copy(idx_src(1), idx1_v, isems[1])

        def step(c2, _):
            for b in range(2):
                c = 2 * c2 + b
                pltpu.make_async_copy(idx_src(c), idxbufs[b], isems[b]).wait()

                @pl.when(c2 >= 1)
                def _():
                    pltpu.make_async_copy(blocks[b], out_dst(c),
                                          ssems[b]).wait()

                compute(b)
                pltpu.async_copy(blocks[b], out_dst(c), ssems[b])

                @pl.when(c2 < _NCH2 - 1)
                def _():
                    pltpu.async_copy(idx_src(c + 2), idxbufs[b], isems[b])
            return 0

        lax.fori_loop(0, _NCH2, step, 0)
        pltpu.make_async_copy(blocks[0], out_dst(0), ssems[0]).wait()
        pltpu.make_async_copy(blocks[1], out_dst(1), ssems[1]).wait()

    return gather_kernel


_sc_gather = _make_sc_gather()


def kernel(indices, table):
    flat_idx = indices.reshape(-1).astype(jnp.int32)
    out_t = _sc_gather(table, flat_idx)
    return out_t.T
